# baseline (speedup 1.0000x reference)
import functools

import jax
import jax.numpy as jnp
from jax import lax
from jax.experimental import pallas as pl
from jax.experimental.pallas import tpu as pltpu
from jax.experimental.pallas import tpu_sc as plsc

B = 64
H = 64
NOPE = 128
ROPE = 64
KV_LORA = 512
V_HEAD = 128
KV_LEN = 32768
TOPK = 2048
DKV = KV_LORA + ROPE
DQ = 768
PK = DQ // 2
SCALE = float((NOPE + ROPE) ** -0.5)

NC = 2
NS = 16
NW = NC * NS
G = 4
BG = B // G
ROWS = BG * TOPK
RPW = ROWS // NW
CH = 128
NCHUNK = RPW // CH


@functools.cache
def _make_sc_gather():
    mesh = plsc.VectorSubcoreMesh(core_axis_name="c", subcore_axis_name="s")

    @functools.partial(
        pl.kernel,
        mesh=mesh,
        out_type=jax.ShapeDtypeStruct((ROWS, PK), jnp.int32),
        scratch_types=[
            pltpu.VMEM((RPW,), jnp.int32),
            pltpu.VMEM((2, CH, PK), jnp.int32),
            pltpu.SemaphoreType.DMA,
            pltpu.SemaphoreType.DMA,
        ],
    )
    def _sc_gather(table_hbm, idx_hbm, out_hbm, idx_v, rows_v, sem0, sem1):
        wid = lax.axis_index("s") * NC + lax.axis_index("c")
        base = wid * RPW
        pltpu.sync_copy(idx_hbm.at[pl.ds(base, RPW)], idx_v)

        def start(c, buf, sem):
            pltpu.async_copy(
                table_hbm.at[idx_v.at[pl.ds(c * CH, CH)]], rows_v.at[buf], sem)

        def wait(buf, sem):
            pltpu.make_async_copy(
                table_hbm.at[idx_v.at[pl.ds(0, CH)]], rows_v.at[buf], sem).wait()

        def store(c, buf):
            pltpu.sync_copy(rows_v.at[buf], out_hbm.at[pl.ds(base + c * CH, CH)])

        start(0, 0, sem0)

        def body(p, carry):
            c0 = 2 * p
            c1 = c0 + 1
            start(c1, 1, sem1)
            wait(0, sem0)
            store(c0, 0)

            @pl.when(c1 + 1 < NCHUNK)
            def _():
                start(c1 + 1, 0, sem0)

            wait(1, sem1)
            store(c1, 1)
            return carry

        lax.fori_loop(0, NCHUNK // 2, body, 0)

    return _sc_gather


HB = 8


def _qproj_body(q_ref, wuk_ref, qf_ref):
    for hh in range(HB):
        qn = q_ref[:, hh, :NOPE]
        qp = q_ref[:, hh, NOPE:]
        ql = lax.dot_general(qn, wuk_ref[hh], (((1,), (0,)), ((), ())),
                             preferred_element_type=jnp.float32)
        qf_ref[:, hh, :KV_LORA] = (ql * SCALE).astype(jnp.bfloat16)
        qf_ref[:, hh, KV_LORA:DKV] = (qp * SCALE).astype(jnp.bfloat16)
        qf_ref[:, hh, DKV:] = jnp.zeros((B, DQ - DKV), jnp.bfloat16)


_qproj = pl.pallas_call(
    _qproj_body,
    grid=(H // HB,),
    in_specs=[
        pl.BlockSpec((B, HB, NOPE + ROPE), lambda h: (0, h, 0)),
        pl.BlockSpec((HB, NOPE, KV_LORA), lambda h: (h, 0, 0)),
    ],
    out_specs=pl.BlockSpec((B, HB, DQ), lambda h: (0, h, 0)),
    out_shape=jax.ShapeDtypeStruct((B, H, DQ), jnp.bfloat16),
)


def _attn_body(qf_ref, sink_ref, kv_ref, o_ref):
    qf = qf_ref[...]
    kw = kv_ref[...]
    k_lo = lax.bitcast_convert_type(
        lax.shift_left(kw, 16), jnp.float32).astype(jnp.bfloat16)
    k_hi = lax.bitcast_convert_type(
        jnp.bitwise_and(kw, jnp.int32(-65536)), jnp.float32).astype(jnp.bfloat16)
    q_lo = qf[:, :PK]
    q_hi = qf[:, PK:]
    logits = (lax.dot_general(q_lo, k_lo, (((1,), (1,)), ((), ())),
                              preferred_element_type=jnp.float32)
              + lax.dot_general(q_hi, k_hi, (((1,), (1,)), ((), ())),
                                preferred_element_type=jnp.float32))
    sink = sink_ref[...]
    m = jnp.maximum(jnp.max(logits, axis=1, keepdims=True), sink)
    p = jnp.exp(logits - m)
    l = jnp.sum(p, axis=1, keepdims=True) + jnp.exp(sink - m)
    rinv = 1.0 / l
    pb = p.astype(jnp.bfloat16)
    o1 = lax.dot_general(pb, k_lo, (((1,), (0,)), ((), ())),
                         preferred_element_type=jnp.float32)
    o2 = lax.dot_general(pb, k_hi[:, :KV_LORA - PK], (((1,), (0,)), ((), ())),
                         preferred_element_type=jnp.float32)
    o_ref[:, :PK] = o1 * rinv
    o_ref[:, PK:] = o2 * rinv


def _make_attn(g):
    return pl.pallas_call(
        _attn_body,
        grid=(BG,),
        in_specs=[
            pl.BlockSpec((None, H, DQ), lambda b: (g * BG + b, 0, 0)),
            pl.BlockSpec((H, 1), lambda b: (0, 0)),
            pl.BlockSpec((None, TOPK, PK), lambda b: (b, 0, 0)),
        ],
        out_specs=pl.BlockSpec((None, H, KV_LORA), lambda b: (b, 0, 0)),
        out_shape=jax.ShapeDtypeStruct((BG, H, KV_LORA), jnp.float32),
    )


def _oproj_body(o_ref, wuv_ref, out_ref):
    for hh in range(HB):
        out_ref[:, hh, :] = lax.dot_general(
            o_ref[:, hh, :], wuv_ref[hh], (((1,), (0,)), ((), ())),
            preferred_element_type=jnp.float32)


_oproj = pl.pallas_call(
    _oproj_body,
    grid=(H // HB,),
    in_specs=[
        pl.BlockSpec((B, HB, KV_LORA), lambda h: (0, h, 0)),
        pl.BlockSpec((HB, KV_LORA, V_HEAD), lambda h: (h, 0, 0)),
    ],
    out_specs=pl.BlockSpec((B, HB, V_HEAD), lambda h: (0, h, 0)),
    out_shape=jax.ShapeDtypeStruct((B, H, V_HEAD), jnp.float32),
)


PBLK = 2048


def _pack_body(kv_ref, out_ref):
    x = kv_ref[...]
    lo_b = x[:, :PK].astype(jnp.bfloat16).astype(jnp.float32)
    hi_b = x[:, PK:].astype(jnp.bfloat16).astype(jnp.float32)
    lo_u = lax.bitcast_convert_type(lo_b, jnp.uint32) >> 16
    hi_u = lax.bitcast_convert_type(hi_b, jnp.uint32) & jnp.uint32(0xFFFF0000)
    hi_full = jnp.concatenate(
        [hi_u, jnp.zeros((PBLK, DQ - DKV), jnp.uint32)], axis=1)
    out_ref[...] = lax.bitcast_convert_type(lo_u | hi_full, jnp.int32)


_pack_cache = pl.pallas_call(
    _pack_body,
    grid=(KV_LEN // PBLK,),
    in_specs=[pl.BlockSpec((PBLK, DKV), lambda i: (i, 0))],
    out_specs=pl.BlockSpec((PBLK, PK), lambda i: (i, 0)),
    out_shape=jax.ShapeDtypeStruct((KV_LEN, PK), jnp.int32),
)


def kernel(q, kv_cache, W_UK, W_UV, attn_sink, topk_indices):
    idx = topk_indices.astype(jnp.int32).reshape(G, ROWS)
    table = _pack_cache(kv_cache)
    q_full = _qproj(q, W_UK)
    sink2 = attn_sink.reshape(H, 1)
    gather = _make_sc_gather()
    kv_parts = [gather(table, idx[g]).reshape(BG, TOPK, PK) for g in range(G)]
    o_parts = [_make_attn(g)(q_full, sink2, kv_parts[g]) for g in range(G)]
    o_lat = jnp.concatenate(o_parts, axis=0)
    out = _oproj(o_lat, W_UV)
    return out.reshape(B, H * V_HEAD)


# speedup vs baseline: 1.1201x; 1.1201x over previous
import functools

import jax
import jax.numpy as jnp
from jax import lax
from jax.experimental import pallas as pl
from jax.experimental.pallas import tpu as pltpu
from jax.experimental.pallas import tpu_sc as plsc

B = 64
H = 64
NOPE = 128
ROPE = 64
KV_LORA = 512
V_HEAD = 128
KV_LEN = 32768
TOPK = 2048
DKV = KV_LORA + ROPE
DQ = 768
PK = DQ // 2
SCALE = float((NOPE + ROPE) ** -0.5)

NC = 2
NS = 16
NW = NC * NS
G = 4
BG = B // G
ROWS = BG * TOPK
RPW = ROWS // NW
CH = 128
NCHUNK = RPW // CH


@functools.cache
def _make_sc_gather():
    mesh = plsc.VectorSubcoreMesh(core_axis_name="c", subcore_axis_name="s")

    @functools.partial(
        pl.kernel,
        mesh=mesh,
        out_type=jax.ShapeDtypeStruct((ROWS, PK), jnp.int32),
        scratch_types=[
            pltpu.VMEM((RPW,), jnp.int32),
            pltpu.VMEM((2, CH, PK), jnp.int32),
            pltpu.SemaphoreType.DMA,
            pltpu.SemaphoreType.DMA,
        ],
    )
    def _sc_gather(table_hbm, idx_hbm, out_hbm, idx_v, rows_v, sem0, sem1):
        wid = lax.axis_index("s") * NC + lax.axis_index("c")
        base = wid * RPW
        pltpu.sync_copy(idx_hbm.at[pl.ds(base, RPW)], idx_v)

        def start(c, buf, sem):
            pltpu.async_copy(
                table_hbm.at[idx_v.at[pl.ds(c * CH, CH)]], rows_v.at[buf], sem)

        def wait(buf, sem):
            pltpu.make_async_copy(
                table_hbm.at[idx_v.at[pl.ds(0, CH)]], rows_v.at[buf], sem).wait()

        def store(c, buf):
            pltpu.sync_copy(rows_v.at[buf], out_hbm.at[pl.ds(base + c * CH, CH)])

        start(0, 0, sem0)

        def body(p, carry):
            c0 = 2 * p
            c1 = c0 + 1
            start(c1, 1, sem1)
            wait(0, sem0)
            store(c0, 0)

            @pl.when(c1 + 1 < NCHUNK)
            def _():
                start(c1 + 1, 0, sem0)

            wait(1, sem1)
            store(c1, 1)
            return carry

        lax.fori_loop(0, NCHUNK // 2, body, 0)

    return _sc_gather


HB = 8


def _qproj_body(q_ref, wuk_ref, qf_ref):
    for hh in range(HB):
        qn = q_ref[:, hh, :NOPE]
        qp = q_ref[:, hh, NOPE:]
        ql = lax.dot_general(qn, wuk_ref[hh], (((1,), (0,)), ((), ())),
                             preferred_element_type=jnp.float32)
        qf_ref[:, hh, :KV_LORA] = ql * SCALE
        qf_ref[:, hh, KV_LORA:DKV] = qp * SCALE
        qf_ref[:, hh, DKV:] = jnp.zeros((B, DQ - DKV), jnp.float32)


_qproj = pl.pallas_call(
    _qproj_body,
    grid=(H // HB,),
    in_specs=[
        pl.BlockSpec((B, HB, NOPE + ROPE), lambda h: (0, h, 0)),
        pl.BlockSpec((HB, NOPE, KV_LORA), lambda h: (h, 0, 0)),
    ],
    out_specs=pl.BlockSpec((B, HB, DQ), lambda h: (0, h, 0)),
    out_shape=jax.ShapeDtypeStruct((B, H, DQ), jnp.float32),
)


def _attn_body(qf_ref, sink_ref, kv_ref, o_ref):
    qf = qf_ref[...].astype(jnp.bfloat16)
    kw = kv_ref[...]
    k_lo = lax.bitcast_convert_type(
        lax.shift_left(kw, 16), jnp.float32).astype(jnp.bfloat16)
    k_hi = lax.bitcast_convert_type(
        jnp.bitwise_and(kw, jnp.int32(-65536)), jnp.float32).astype(jnp.bfloat16)
    q_lo = qf[:, :PK]
    q_hi = qf[:, PK:]
    logits = (lax.dot_general(q_lo, k_lo, (((1,), (1,)), ((), ())),
                              preferred_element_type=jnp.float32)
              + lax.dot_general(q_hi, k_hi, (((1,), (1,)), ((), ())),
                                preferred_element_type=jnp.float32))
    sink = sink_ref[...]
    m = jnp.maximum(jnp.max(logits, axis=1, keepdims=True), sink)
    p = jnp.exp(logits - m)
    l = jnp.sum(p, axis=1, keepdims=True) + jnp.exp(sink - m)
    rinv = 1.0 / l
    pb = p.astype(jnp.bfloat16)
    o1 = lax.dot_general(pb, k_lo, (((1,), (0,)), ((), ())),
                         preferred_element_type=jnp.float32)
    o2 = lax.dot_general(pb, k_hi[:, :KV_LORA - PK], (((1,), (0,)), ((), ())),
                         preferred_element_type=jnp.float32)
    o_ref[:, :PK] = o1 * rinv
    o_ref[:, PK:] = o2 * rinv


def _make_attn(g):
    return pl.pallas_call(
        _attn_body,
        grid=(BG,),
        in_specs=[
            pl.BlockSpec((None, H, DQ), lambda b: (g * BG + b, 0, 0)),
            pl.BlockSpec((H, 1), lambda b: (0, 0)),
            pl.BlockSpec((None, TOPK, PK), lambda b: (b, 0, 0)),
        ],
        out_specs=pl.BlockSpec((None, H, KV_LORA), lambda b: (b, 0, 0)),
        out_shape=jax.ShapeDtypeStruct((BG, H, KV_LORA), jnp.float32),
    )


def _oproj_body(o_ref, wuv_ref, out_ref):
    for hh in range(HB):
        out_ref[:, hh, :] = lax.dot_general(
            o_ref[:, hh, :], wuv_ref[hh], (((1,), (0,)), ((), ())),
            preferred_element_type=jnp.float32)


_oproj = pl.pallas_call(
    _oproj_body,
    grid=(H // HB,),
    in_specs=[
        pl.BlockSpec((B, HB, KV_LORA), lambda h: (0, h, 0)),
        pl.BlockSpec((HB, KV_LORA, V_HEAD), lambda h: (h, 0, 0)),
    ],
    out_specs=pl.BlockSpec((B, HB, V_HEAD), lambda h: (0, h, 0)),
    out_shape=jax.ShapeDtypeStruct((B, H, V_HEAD), jnp.float32),
)


PBLK = 512


def _pack_body(kvt_ref, out_ref):
    x = kvt_ref[...]
    lo_b = x[:PK, :].astype(jnp.bfloat16).astype(jnp.float32)
    hi_b = x[PK:, :].astype(jnp.bfloat16).astype(jnp.float32)
    lo_u = lax.bitcast_convert_type(lo_b, jnp.uint32) >> 16
    hi_u = lax.bitcast_convert_type(hi_b, jnp.uint32) & jnp.uint32(0xFFFF0000)
    hi_full = jnp.concatenate(
        [hi_u, jnp.zeros((DQ - DKV, PBLK), jnp.uint32)], axis=0)
    wt = lax.bitcast_convert_type(lo_u | hi_full, jnp.int32)
    out_ref[...] = lax.transpose(wt, (1, 0))


_pack_cache = pl.pallas_call(
    _pack_body,
    grid=(KV_LEN // PBLK,),
    in_specs=[pl.BlockSpec((DKV, PBLK), lambda i: (0, i))],
    out_specs=pl.BlockSpec((PBLK, PK), lambda i: (i, 0)),
    out_shape=jax.ShapeDtypeStruct((KV_LEN, PK), jnp.int32),
)


def kernel(q, kv_cache, W_UK, W_UV, attn_sink, topk_indices):
    idx = topk_indices.astype(jnp.int32).reshape(G, ROWS)
    table = _pack_cache(kv_cache.T)
    q_full = _qproj(q, W_UK)
    sink2 = attn_sink.reshape(H, 1)
    gather = _make_sc_gather()
    kv_parts = [gather(table, idx[g]).reshape(BG, TOPK, PK) for g in range(G)]
    o_parts = [_make_attn(g)(q_full, sink2, kv_parts[g]) for g in range(G)]
    o_lat = jnp.concatenate(o_parts, axis=0)
    out = _oproj(o_lat, W_UV)
    return out.reshape(B, H * V_HEAD)


# speedup vs baseline: 1.2160x; 1.0856x over previous
import functools

import jax
import jax.numpy as jnp
from jax import lax
from jax.experimental import pallas as pl
from jax.experimental.pallas import tpu as pltpu
from jax.experimental.pallas import tpu_sc as plsc

B = 64
H = 64
NOPE = 128
ROPE = 64
KV_LORA = 512
V_HEAD = 128
KV_LEN = 32768
TOPK = 2048
DKV = KV_LORA + ROPE
DQ = 768
PK = DQ // 2
SCALE = float((NOPE + ROPE) ** -0.5)

NC = 2
NS = 16
NW = NC * NS
G = 4
BG = B // G
ROWS = BG * TOPK
RPW = ROWS // NW
CH = 128
NCHUNK = RPW // CH


@functools.cache
def _make_sc_gather():
    mesh = plsc.VectorSubcoreMesh(core_axis_name="c", subcore_axis_name="s")

    @functools.partial(
        pl.kernel,
        mesh=mesh,
        out_type=jax.ShapeDtypeStruct((ROWS, PK), jnp.int32),
        scratch_types=[
            pltpu.VMEM((RPW,), jnp.int32),
            pltpu.VMEM((2, CH, PK), jnp.int32),
            pltpu.SemaphoreType.DMA,
            pltpu.SemaphoreType.DMA,
        ],
    )
    def _sc_gather(table_hbm, idx_hbm, out_hbm, idx_v, rows_v, sem0, sem1):
        wid = lax.axis_index("s") * NC + lax.axis_index("c")
        base = wid * RPW
        pltpu.sync_copy(idx_hbm.at[pl.ds(base, RPW)], idx_v)

        def start(c, buf, sem):
            pltpu.async_copy(
                table_hbm.at[idx_v.at[pl.ds(c * CH, CH)]], rows_v.at[buf], sem)

        def wait(buf, sem):
            pltpu.make_async_copy(
                table_hbm.at[idx_v.at[pl.ds(0, CH)]], rows_v.at[buf], sem).wait()

        def store(c, buf):
            pltpu.sync_copy(rows_v.at[buf], out_hbm.at[pl.ds(base + c * CH, CH)])

        start(0, 0, sem0)

        def body(p, carry):
            c0 = 2 * p
            c1 = c0 + 1
            start(c1, 1, sem1)
            wait(0, sem0)
            store(c0, 0)

            @pl.when(c1 + 1 < NCHUNK)
            def _():
                start(c1 + 1, 0, sem0)

            wait(1, sem1)
            store(c1, 1)
            return carry

        lax.fori_loop(0, NCHUNK // 2, body, 0)

    return _sc_gather


HB = 8


def _qproj_body(q_ref, wuk_ref, qf_ref):
    for hh in range(HB):
        qn = q_ref[:, hh, :NOPE]
        qp = q_ref[:, hh, NOPE:]
        ql = lax.dot_general(qn, wuk_ref[hh], (((1,), (0,)), ((), ())),
                             preferred_element_type=jnp.float32)
        qf_ref[:, hh, :KV_LORA] = ql * SCALE
        qf_ref[:, hh, KV_LORA:DKV] = qp * SCALE
        qf_ref[:, hh, DKV:] = jnp.zeros((B, DQ - DKV), jnp.float32)


_qproj = pl.pallas_call(
    _qproj_body,
    grid=(H // HB,),
    in_specs=[
        pl.BlockSpec((B, HB, NOPE + ROPE), lambda h: (0, h, 0)),
        pl.BlockSpec((HB, NOPE, KV_LORA), lambda h: (h, 0, 0)),
    ],
    out_specs=pl.BlockSpec((B, HB, DQ), lambda h: (0, h, 0)),
    out_shape=jax.ShapeDtypeStruct((B, H, DQ), jnp.float32),
)


def _attn_body(qf_ref, sink_ref, kv0_ref, kv1_ref, o_ref):
    qf = qf_ref[...].astype(jnp.bfloat16)
    kw0 = kv0_ref[...]
    kw1 = kv1_ref[...]
    k0_lo = lax.bitcast_convert_type(
        lax.shift_left(kw0, 16), jnp.float32).astype(jnp.bfloat16)
    k0_hi = lax.bitcast_convert_type(
        jnp.bitwise_and(kw0, jnp.int32(-65536)),
        jnp.float32).astype(jnp.bfloat16)
    k1_lo = lax.bitcast_convert_type(
        lax.shift_left(kw1, 16), jnp.float32).astype(jnp.bfloat16)
    logits = (lax.dot_general(qf[:, :256], k0_lo, (((1,), (1,)), ((), ())),
                              preferred_element_type=jnp.float32)
              + lax.dot_general(qf[:, 256:PK], k1_lo, (((1,), (1,)), ((), ())),
                                preferred_element_type=jnp.float32)
              + lax.dot_general(qf[:, PK:640], k0_hi, (((1,), (1,)), ((), ())),
                                preferred_element_type=jnp.float32))
    sink = sink_ref[...]
    m = jnp.maximum(jnp.max(logits, axis=1, keepdims=True), sink)
    p = jnp.exp(logits - m)
    l = jnp.sum(p, axis=1, keepdims=True) + jnp.exp(sink - m)
    rinv = 1.0 / l
    pb = p.astype(jnp.bfloat16)
    o0 = lax.dot_general(pb, k0_lo, (((1,), (0,)), ((), ())),
                         preferred_element_type=jnp.float32)
    o1 = lax.dot_general(pb, k1_lo, (((1,), (0,)), ((), ())),
                         preferred_element_type=jnp.float32)
    o2 = lax.dot_general(pb, k0_hi[:, :KV_LORA - PK], (((1,), (0,)), ((), ())),
                         preferred_element_type=jnp.float32)
    o_ref[:, :256] = o0 * rinv
    o_ref[:, 256:PK] = o1 * rinv
    o_ref[:, PK:] = o2 * rinv


def _make_attn(g):
    return pl.pallas_call(
        _attn_body,
        grid=(BG,),
        in_specs=[
            pl.BlockSpec((None, H, DQ), lambda b: (g * BG + b, 0, 0)),
            pl.BlockSpec((H, 1), lambda b: (0, 0)),
            pl.BlockSpec((None, TOPK, 256), lambda b: (b, 0, 0)),
            pl.BlockSpec((None, TOPK, 128), lambda b: (b, 0, 2)),
        ],
        out_specs=pl.BlockSpec((None, H, KV_LORA), lambda b: (b, 0, 0)),
        out_shape=jax.ShapeDtypeStruct((BG, H, KV_LORA), jnp.float32),
    )


def _oproj_body(o_ref, wuv_ref, out_ref):
    for hh in range(HB):
        out_ref[:, hh, :] = lax.dot_general(
            o_ref[:, hh, :], wuv_ref[hh], (((1,), (0,)), ((), ())),
            preferred_element_type=jnp.float32)


_oproj = pl.pallas_call(
    _oproj_body,
    grid=(H // HB,),
    in_specs=[
        pl.BlockSpec((B, HB, KV_LORA), lambda h: (0, h, 0)),
        pl.BlockSpec((HB, KV_LORA, V_HEAD), lambda h: (h, 0, 0)),
    ],
    out_specs=pl.BlockSpec((B, HB, V_HEAD), lambda h: (0, h, 0)),
    out_shape=jax.ShapeDtypeStruct((B, H, V_HEAD), jnp.float32),
)


PBLK = 2048


def _pack_body(kvt_ref, out_ref):
    x = kvt_ref[...]
    lo_b = x[:PK, :].astype(jnp.bfloat16).astype(jnp.float32)
    hi_b = x[PK:, :].astype(jnp.bfloat16).astype(jnp.float32)
    lo_u = lax.bitcast_convert_type(lo_b, jnp.uint32) >> 16
    hi_u = lax.bitcast_convert_type(hi_b, jnp.uint32) & jnp.uint32(0xFFFF0000)
    hi_full = jnp.concatenate(
        [hi_u, jnp.zeros((DQ - DKV, PBLK), jnp.uint32)], axis=0)
    wt = lax.bitcast_convert_type(lo_u | hi_full, jnp.int32)
    out_ref[...] = lax.transpose(wt, (1, 0))


_pack_cache = pl.pallas_call(
    _pack_body,
    grid=(KV_LEN // PBLK,),
    in_specs=[pl.BlockSpec((DKV, PBLK), lambda i: (0, i))],
    out_specs=pl.BlockSpec((PBLK, PK), lambda i: (i, 0)),
    out_shape=jax.ShapeDtypeStruct((KV_LEN, PK), jnp.int32),
)


def kernel(q, kv_cache, W_UK, W_UV, attn_sink, topk_indices):
    idx = topk_indices.astype(jnp.int32).reshape(G, ROWS)
    table = _pack_cache(kv_cache.T)
    q_full = _qproj(q, W_UK)
    sink2 = attn_sink.reshape(H, 1)
    gather = _make_sc_gather()
    kv_parts = [gather(table, idx[g]).reshape(BG, TOPK, PK) for g in range(G)]
    o_parts = [_make_attn(g)(q_full, sink2, kv_parts[g], kv_parts[g])
               for g in range(G)]
    o_lat = jnp.concatenate(o_parts, axis=0)
    out = _oproj(o_lat, W_UV)
    return out.reshape(B, H * V_HEAD)
